# baseline (speedup 1.0000x reference)
import functools

import jax
import jax.numpy as jnp
from jax import lax
from jax.experimental import pallas as pl
from jax.experimental.pallas import tpu as pltpu
from jax.experimental.pallas import tpu_sc as plsc

N = 10000
E = 40000
D = 256
H = 128
NS = 16
NC = 2
EPT = 2560
EPAD = NS * EPT
CH = 64
NCHUNK = EPT // CH
ACC_ROWS = 10112
STRIPE = ACC_ROWS // NS
TRASH = 10100
EPT_CNT = EPAD // (NC * NS)
CNT_CH = 128

_MESH = plsc.VectorSubcoreMesh(core_axis_name="c", subcore_axis_name="s")


@functools.partial(
    pl.kernel,
    out_type=tuple([jax.ShapeDtypeStruct((NC, ACC_ROWS, H), jnp.float32)] * 4),
    mesh=_MESH,
    scratch_types=[
        pltpu.VMEM_SHARED((ACC_ROWS, H), jnp.float32),
        pltpu.VMEM((NCHUNK + 1, CH), jnp.int32),
        pltpu.VMEM((NCHUNK + 1, CH), jnp.int32),
        pltpu.VMEM((CH, H), jnp.float32),
        pltpu.VMEM((CH, H), jnp.float32),
        pltpu.SemaphoreType.DMA,
        pltpu.SemaphoreType.DMA,
    ])
def _sc_agg(tab_c, tab_d,
            src_cause, dst_cause, src_relate, dst_relate,
            src_child, dst_child, src_rev, dst_rev,
            zrow_hbm,
            out_cause, out_relate, out_child, out_rev,
            acc_sh, sidx_st, didx_st, rows_a, rows_b, sem_a, sem_b):
    tile = lax.axis_index("s")
    cid = lax.axis_index("c")

    rels = (
        (src_cause, dst_cause, tab_c, out_cause),
        (src_relate, dst_relate, tab_c, out_relate),
        (src_child, dst_child, tab_d, out_child),
        (src_rev, dst_rev, tab_d, out_rev),
    )
    for (src3, dst3, tab, agg_out) in rels:
        pltpu.sync_copy(zrow_hbm, acc_sh.at[pl.ds(tile * STRIPE, STRIPE)])
        pltpu.sync_copy(src3.at[cid, tile], sidx_st)
        pltpu.sync_copy(dst3.at[tile], didx_st)
        plsc.subcore_barrier()

        pltpu.async_copy(tab.at[sidx_st.at[0]], rows_a, sem_a)

        def pair(t, carry):
            j0 = 2 * t
            pltpu.make_async_copy(tab.at[sidx_st.at[j0]], rows_a,
                                  sem_a).wait()
            pltpu.async_copy(tab.at[sidx_st.at[j0 + 1]], rows_b, sem_b)
            pltpu.sync_copy(rows_a, acc_sh.at[didx_st.at[j0]], add=True)
            pltpu.make_async_copy(tab.at[sidx_st.at[j0 + 1]], rows_b,
                                  sem_b).wait()
            nxt = jnp.minimum(j0 + 2, NCHUNK)
            pltpu.async_copy(tab.at[sidx_st.at[nxt]], rows_a, sem_a)
            pltpu.sync_copy(rows_b, acc_sh.at[didx_st.at[j0 + 1]], add=True)
            return carry

        lax.fori_loop(0, NCHUNK // 2, pair, 0)
        pltpu.make_async_copy(tab.at[sidx_st.at[NCHUNK]], rows_a,
                              sem_a).wait()
        plsc.subcore_barrier()
        pltpu.sync_copy(acc_sh.at[pl.ds(tile * STRIPE, STRIPE)],
                        agg_out.at[cid, pl.ds(tile * STRIPE, STRIPE)])


@functools.partial(
    pl.kernel,
    out_type=tuple([jax.ShapeDtypeStruct((NC, ACC_ROWS, H), jnp.float32)] * 4),
    mesh=_MESH,
    scratch_types=[
        pltpu.VMEM_SHARED((ACC_ROWS, H), jnp.float32),
        pltpu.VMEM((CNT_CH,), jnp.int32),
        pltpu.VMEM((CNT_CH, H), jnp.float32),
    ])
def _sc_counts(dst_cause, dst_relate, dst_child, dst_rev,
               zrow_hbm, ones_hbm,
               out_cause, out_relate, out_child, out_rev,
               cnt_sh, didx_v, ones_v):
    tile = lax.axis_index("s")
    cid = lax.axis_index("c")
    base = cid * (EPAD // NC) + tile * EPT_CNT

    pltpu.sync_copy(ones_hbm, ones_v)
    for (dst, cnt_out) in ((dst_cause, out_cause), (dst_relate, out_relate),
                           (dst_child, out_child), (dst_rev, out_rev)):
        pltpu.sync_copy(zrow_hbm, cnt_sh.at[pl.ds(tile * STRIPE, STRIPE)])
        plsc.subcore_barrier()

        def chunk(j, carry):
            pltpu.sync_copy(dst.at[pl.ds(base + j * CNT_CH, CNT_CH)], didx_v)
            pltpu.sync_copy(ones_v, cnt_sh.at[didx_v], add=True)
            return carry

        lax.fori_loop(0, EPT_CNT // CNT_CH, chunk, 0)
        plsc.subcore_barrier()
        pltpu.sync_copy(cnt_sh.at[pl.ds(tile * STRIPE, STRIPE)],
                        cnt_out.at[cid, pl.ds(tile * STRIPE, STRIPE)])


def _dot_t(x, w):
    return lax.dot_general(x, w, (((1,), (1,)), ((), ())),
                           preferred_element_type=jnp.float32)


def _tc_layer_body(relu, halves,
                   aggc, cntc_a, cntc_b, wlc, blc,
                   aggr, cntr_a, cntr_b, wlr, blr,
                   aggch, cntch_a, cntch_b, wlch, blch,
                   wrc, wrr, wrch, xd2,
                   aggv, cntv_a, cntv_b, wlv, blv, wrv, xc2,
                   outc_ref, outd_ref):
    def sage(agg_ref, cnt_a, cnt_b, wl_ref):
        cnt = cnt_a[:, 0:1] + cnt_b[:, 0:1]
        inv = 1.0 / jnp.maximum(cnt, 1.0)
        lo = agg_ref[0] * inv
        hi = agg_ref[1] * inv
        return _dot_t(lo, wl_ref[:, 0:H]) + _dot_t(hi, wl_ref[:, H:D])

    def dense(x_ref, w):
        return _dot_t(x_ref[0], w[:, 0:H]) + _dot_t(x_ref[1], w[:, H:D])

    outd = (sage(aggc, cntc_a, cntc_b, wlc)
            + sage(aggr, cntr_a, cntr_b, wlr)
            + sage(aggch, cntch_a, cntch_b, wlch)
            + dense(xd2, wrc[...] + wrr[...] + wrch[...])
            + (blc[...] + blr[...] + blch[...]))
    outc = sage(aggv, cntv_a, cntv_b, wlv) + dense(xc2, wrv[...]) + blv[...]
    if relu:
        outd = jnp.maximum(outd, 0.0)
        outc = jnp.maximum(outc, 0.0)
    if halves:
        outc_ref[0] = outc[:, 0:H]
        outc_ref[1] = outc[:, H:D]
        outd_ref[0] = outd[:, 0:H]
        outd_ref[1] = outd[:, H:D]
    else:
        outc_ref[...] = outc
        outd_ref[...] = outd


def _tc_layer_call(relu, halves, args):
    MB = 400
    grid = (N // MB,)
    agg_spec = pl.BlockSpec((NC, MB, H), lambda i: (0, i, 0))
    cnt_spec = pl.BlockSpec((MB, 16), lambda i: (i, 0))
    w_spec = pl.BlockSpec((D, D), lambda i: (0, 0))
    b_spec = pl.BlockSpec((1, D), lambda i: (0, 0))
    x_spec = pl.BlockSpec((NC, MB, H), lambda i: (0, i, 0))
    rel_spec = [agg_spec, cnt_spec, cnt_spec, w_spec, b_spec]
    in_specs = (rel_spec * 3
                + [w_spec, w_spec, w_spec, x_spec]
                + rel_spec + [w_spec, x_spec])
    if halves:
        out_shape = [jax.ShapeDtypeStruct((NC, N, H), jnp.float32)] * 2
        out_specs = [pl.BlockSpec((NC, MB, H), lambda i: (0, i, 0))] * 2
    else:
        out_shape = [jax.ShapeDtypeStruct((N, D), jnp.float32)] * 2
        out_specs = [pl.BlockSpec((MB, D), lambda i: (i, 0))] * 2
    fn = pl.pallas_call(
        functools.partial(_tc_layer_body, relu, halves),
        grid=grid, in_specs=in_specs, out_specs=out_specs,
        out_shape=out_shape)
    return fn(*args)


def kernel(x_chemical, x_disease,
           edge_index_cause, edge_index_relate, edge_index_rev,
           edge_index_child,
           Wl_0_cause, bl_0_cause, Wr_0_cause,
           Wl_0_relate, bl_0_relate, Wr_0_relate,
           Wl_0_rev, bl_0_rev, Wr_0_rev,
           Wl_0_child, bl_0_child, Wr_0_child,
           Wl_1_cause, bl_1_cause, Wr_1_cause,
           Wl_1_relate, bl_1_relate, Wr_1_relate,
           Wl_1_rev, bl_1_rev, Wr_1_rev,
           Wl_1_child, bl_1_child, Wr_1_child):
    def stack_halves(x):
        return x.reshape(N, NC, H).transpose(1, 0, 2)

    xc2 = stack_halves(x_chemical)
    xd2 = stack_halves(x_disease)

    def prep_edges(ei):
        src = ei[0].astype(jnp.int32)
        dst = ei[1].astype(jnp.int32)
        pad = EPAD - E
        src = jnp.concatenate([src, jnp.zeros((pad,), jnp.int32)])
        dst = jnp.concatenate([dst, jnp.full((pad,), TRASH, jnp.int32)])
        zc = jnp.zeros((NS, 1, CH), jnp.int32)
        src3 = src.reshape(NS, NCHUNK, CH)
        src3 = jnp.concatenate([src3, zc], axis=1)
        src3 = jnp.stack([src3, src3 + N])
        dst3 = jnp.concatenate([dst.reshape(NS, NCHUNK, CH), zc], axis=1)
        return src3, dst3, dst

    sc_c, dc3, dc = prep_edges(edge_index_cause)
    sc_r, dr3, dr = prep_edges(edge_index_relate)
    sc_ch, dch3, dch = prep_edges(edge_index_child)
    sc_v, dv3, dv = prep_edges(edge_index_rev)

    zrow = jnp.zeros((STRIPE, H), jnp.float32)
    ones = jnp.ones((CNT_CH, H), jnp.float32)

    biases0 = [b.reshape(1, D) for b in
               (bl_0_cause, bl_0_relate, bl_0_child, bl_0_rev)]
    biases1 = [b.reshape(1, D) for b in
               (bl_1_cause, bl_1_relate, bl_1_child, bl_1_rev)]

    cnt2 = _sc_counts(dc, dr, dch, dv, zrow, ones)
    cnts = [(c2[0, :, 0:16], c2[1, :, 0:16]) for c2 in cnt2]

    (ag_c, ag_r, ag_ch, ag_v) = _sc_agg(
        xc2.reshape(NC * N, H), xd2.reshape(NC * N, H),
        sc_c, dc3, sc_r, dr3, sc_ch, dch3, sc_v, dv3, zrow)

    hc2, hd2 = _tc_layer_call(True, True, (
        ag_c, cnts[0][0], cnts[0][1], Wl_0_cause, biases0[0],
        ag_r, cnts[1][0], cnts[1][1], Wl_0_relate, biases0[1],
        ag_ch, cnts[2][0], cnts[2][1], Wl_0_child, biases0[2],
        Wr_0_cause, Wr_0_relate, Wr_0_child, xd2,
        ag_v, cnts[3][0], cnts[3][1], Wl_0_rev, biases0[3], Wr_0_rev, xc2,
    ))

    (bg_c, bg_r, bg_ch, bg_v) = _sc_agg(
        hc2.reshape(NC * N, H), hd2.reshape(NC * N, H),
        sc_c, dc3, sc_r, dr3, sc_ch, dch3, sc_v, dv3, zrow)

    out_c, out_d = _tc_layer_call(False, False, (
        bg_c, cnts[0][0], cnts[0][1], Wl_1_cause, biases1[0],
        bg_r, cnts[1][0], cnts[1][1], Wl_1_relate, biases1[1],
        bg_ch, cnts[2][0], cnts[2][1], Wl_1_child, biases1[2],
        Wr_1_cause, Wr_1_relate, Wr_1_child, hd2,
        bg_v, cnts[3][0], cnts[3][1], Wl_1_rev, biases1[3], Wr_1_rev, hc2,
    ))
    return out_c, out_d


# speedup vs baseline: 1.3141x; 1.3141x over previous
import functools

import jax
import jax.numpy as jnp
from jax import lax
from jax.experimental import pallas as pl
from jax.experimental.pallas import tpu as pltpu
from jax.experimental.pallas import tpu_sc as plsc

N = 10000
E = 40000
D = 256
H = 128
NS = 16
NC = 2
EPT = 2560
EPAD = NS * EPT
CH = 128
NCHUNK = EPT // CH
ACC_ROWS = 10112
STRIPE = ACC_ROWS // NS
TRASH = 10100
EPT_CNT = EPAD // (NC * NS)

_MESH = plsc.VectorSubcoreMesh(core_axis_name="c", subcore_axis_name="s")


@functools.partial(
    pl.kernel,
    out_type=tuple([jax.ShapeDtypeStruct((NC, ACC_ROWS, H), jnp.float32)] * 4),
    mesh=_MESH,
    scratch_types=[
        pltpu.VMEM_SHARED((ACC_ROWS, H), jnp.float32),
        pltpu.VMEM((NCHUNK, CH), jnp.int32),
        pltpu.VMEM((NCHUNK, CH), jnp.int32),
        pltpu.VMEM((CH, H), jnp.float32),
        pltpu.SemaphoreType.DMA,
    ])
def _sc_agg(tab_c, tab_d,
            src_cause, dst_cause, src_relate, dst_relate,
            src_child, dst_child, src_rev, dst_rev,
            zrow_hbm,
            out_cause, out_relate, out_child, out_rev,
            acc_sh, sidx_st, didx_st, rows_v, sem):
    tile = lax.axis_index("s")
    cid = lax.axis_index("c")

    rels = (
        (src_cause, dst_cause, tab_c, out_cause),
        (src_relate, dst_relate, tab_c, out_relate),
        (src_child, dst_child, tab_d, out_child),
        (src_rev, dst_rev, tab_d, out_rev),
    )
    for (src3, dst3, tab, agg_out) in rels:
        pltpu.sync_copy(zrow_hbm, acc_sh.at[pl.ds(tile * STRIPE, STRIPE)])
        pltpu.sync_copy(src3.at[cid, tile], sidx_st)
        pltpu.sync_copy(dst3.at[tile], didx_st)
        plsc.subcore_barrier()

        def chunk(j, carry):
            pltpu.async_copy(tab.at[sidx_st.at[j]], rows_v, sem).wait()
            pltpu.sync_copy(rows_v, acc_sh.at[didx_st.at[j]], add=True)
            return carry

        lax.fori_loop(0, NCHUNK, chunk, 0)
        plsc.subcore_barrier()
        pltpu.sync_copy(acc_sh.at[pl.ds(tile * STRIPE, STRIPE)],
                        agg_out.at[cid, pl.ds(tile * STRIPE, STRIPE)])


@functools.partial(
    pl.kernel,
    out_type=tuple([jax.ShapeDtypeStruct((NC, ACC_ROWS, H), jnp.float32)] * 4),
    mesh=_MESH,
    scratch_types=[
        pltpu.VMEM_SHARED((ACC_ROWS, H), jnp.float32),
        pltpu.VMEM((CH,), jnp.int32),
        pltpu.VMEM((CH, H), jnp.float32),
    ])
def _sc_counts(dst_cause, dst_relate, dst_child, dst_rev,
               zrow_hbm, ones_hbm,
               out_cause, out_relate, out_child, out_rev,
               cnt_sh, didx_v, ones_v):
    tile = lax.axis_index("s")
    cid = lax.axis_index("c")
    base = cid * (EPAD // NC) + tile * EPT_CNT

    pltpu.sync_copy(ones_hbm, ones_v)
    for (dst, cnt_out) in ((dst_cause, out_cause), (dst_relate, out_relate),
                           (dst_child, out_child), (dst_rev, out_rev)):
        pltpu.sync_copy(zrow_hbm, cnt_sh.at[pl.ds(tile * STRIPE, STRIPE)])
        plsc.subcore_barrier()

        def chunk(j, carry):
            pltpu.sync_copy(dst.at[pl.ds(base + j * CH, CH)], didx_v)
            pltpu.sync_copy(ones_v, cnt_sh.at[didx_v], add=True)
            return carry

        lax.fori_loop(0, EPT_CNT // CH, chunk, 0)
        plsc.subcore_barrier()
        pltpu.sync_copy(cnt_sh.at[pl.ds(tile * STRIPE, STRIPE)],
                        cnt_out.at[cid, pl.ds(tile * STRIPE, STRIPE)])


def _dot_t(x, w):
    return lax.dot_general(x, w, (((1,), (1,)), ((), ())),
                           preferred_element_type=jnp.float32)


def _tc_layer_body(relu, halves,
                   aggc, cntc_a, cntc_b, wlc, blc,
                   aggr, cntr_a, cntr_b, wlr, blr,
                   aggch, cntch_a, cntch_b, wlch, blch,
                   wrc, wrr, wrch, xd2,
                   aggv, cntv_a, cntv_b, wlv, blv, wrv, xc2,
                   outc_ref, outd_ref):
    def sage(agg_ref, cnt_a, cnt_b, wl_ref):
        cnt = cnt_a[:, 0:1] + cnt_b[:, 0:1]
        inv = 1.0 / jnp.maximum(cnt, 1.0)
        lo = agg_ref[0] * inv
        hi = agg_ref[1] * inv
        return _dot_t(lo, wl_ref[:, 0:H]) + _dot_t(hi, wl_ref[:, H:D])

    def dense(x_ref, w):
        return _dot_t(x_ref[0], w[:, 0:H]) + _dot_t(x_ref[1], w[:, H:D])

    outd = (sage(aggc, cntc_a, cntc_b, wlc)
            + sage(aggr, cntr_a, cntr_b, wlr)
            + sage(aggch, cntch_a, cntch_b, wlch)
            + dense(xd2, wrc[...] + wrr[...] + wrch[...])
            + (blc[...] + blr[...] + blch[...]))
    outc = sage(aggv, cntv_a, cntv_b, wlv) + dense(xc2, wrv[...]) + blv[...]
    if relu:
        outd = jnp.maximum(outd, 0.0)
        outc = jnp.maximum(outc, 0.0)
    if halves:
        outc_ref[0] = outc[:, 0:H]
        outc_ref[1] = outc[:, H:D]
        outd_ref[0] = outd[:, 0:H]
        outd_ref[1] = outd[:, H:D]
    else:
        outc_ref[...] = outc
        outd_ref[...] = outd


def _tc_layer_call(relu, halves, args):
    MB = 400
    grid = (N // MB,)
    agg_spec = pl.BlockSpec((NC, MB, H), lambda i: (0, i, 0))
    cnt_spec = pl.BlockSpec((MB, 16), lambda i: (i, 0))
    w_spec = pl.BlockSpec((D, D), lambda i: (0, 0))
    b_spec = pl.BlockSpec((1, D), lambda i: (0, 0))
    x_spec = pl.BlockSpec((NC, MB, H), lambda i: (0, i, 0))
    rel_spec = [agg_spec, cnt_spec, cnt_spec, w_spec, b_spec]
    in_specs = (rel_spec * 3
                + [w_spec, w_spec, w_spec, x_spec]
                + rel_spec + [w_spec, x_spec])
    if halves:
        out_shape = [jax.ShapeDtypeStruct((NC, N, H), jnp.float32)] * 2
        out_specs = [pl.BlockSpec((NC, MB, H), lambda i: (0, i, 0))] * 2
    else:
        out_shape = [jax.ShapeDtypeStruct((N, D), jnp.float32)] * 2
        out_specs = [pl.BlockSpec((MB, D), lambda i: (i, 0))] * 2
    fn = pl.pallas_call(
        functools.partial(_tc_layer_body, relu, halves),
        grid=grid, in_specs=in_specs, out_specs=out_specs,
        out_shape=out_shape)
    return fn(*args)


def kernel(x_chemical, x_disease,
           edge_index_cause, edge_index_relate, edge_index_rev,
           edge_index_child,
           Wl_0_cause, bl_0_cause, Wr_0_cause,
           Wl_0_relate, bl_0_relate, Wr_0_relate,
           Wl_0_rev, bl_0_rev, Wr_0_rev,
           Wl_0_child, bl_0_child, Wr_0_child,
           Wl_1_cause, bl_1_cause, Wr_1_cause,
           Wl_1_relate, bl_1_relate, Wr_1_relate,
           Wl_1_rev, bl_1_rev, Wr_1_rev,
           Wl_1_child, bl_1_child, Wr_1_child):
    def stack_halves(x):
        return x.reshape(N, NC, H).transpose(1, 0, 2)

    xc2 = stack_halves(x_chemical)
    xd2 = stack_halves(x_disease)

    def prep_edges(ei):
        src = ei[0].astype(jnp.int32)
        dst = ei[1].astype(jnp.int32)
        pad = EPAD - E
        src = jnp.concatenate([src, jnp.zeros((pad,), jnp.int32)])
        dst = jnp.concatenate([dst, jnp.full((pad,), TRASH, jnp.int32)])
        src3 = src.reshape(NS, NCHUNK, CH)
        src3 = jnp.stack([src3, src3 + N])
        dst3 = dst.reshape(NS, NCHUNK, CH)
        return src3, dst3, dst

    sc_c, dc3, dc = prep_edges(edge_index_cause)
    sc_r, dr3, dr = prep_edges(edge_index_relate)
    sc_ch, dch3, dch = prep_edges(edge_index_child)
    sc_v, dv3, dv = prep_edges(edge_index_rev)

    zrow = jnp.zeros((STRIPE, H), jnp.float32)
    ones = jnp.ones((CH, H), jnp.float32)

    biases0 = [b.reshape(1, D) for b in
               (bl_0_cause, bl_0_relate, bl_0_child, bl_0_rev)]
    biases1 = [b.reshape(1, D) for b in
               (bl_1_cause, bl_1_relate, bl_1_child, bl_1_rev)]

    cnt2 = _sc_counts(dc, dr, dch, dv, zrow, ones)
    cnts = [(c2[0, :, 0:16], c2[1, :, 0:16]) for c2 in cnt2]

    (ag_c, ag_r, ag_ch, ag_v) = _sc_agg(
        xc2.reshape(NC * N, H), xd2.reshape(NC * N, H),
        sc_c, dc3, sc_r, dr3, sc_ch, dch3, sc_v, dv3, zrow)

    hc2, hd2 = _tc_layer_call(True, True, (
        ag_c, cnts[0][0], cnts[0][1], Wl_0_cause, biases0[0],
        ag_r, cnts[1][0], cnts[1][1], Wl_0_relate, biases0[1],
        ag_ch, cnts[2][0], cnts[2][1], Wl_0_child, biases0[2],
        Wr_0_cause, Wr_0_relate, Wr_0_child, xd2,
        ag_v, cnts[3][0], cnts[3][1], Wl_0_rev, biases0[3], Wr_0_rev, xc2,
    ))

    (bg_c, bg_r, bg_ch, bg_v) = _sc_agg(
        hc2.reshape(NC * N, H), hd2.reshape(NC * N, H),
        sc_c, dc3, sc_r, dr3, sc_ch, dch3, sc_v, dv3, zrow)

    out_c, out_d = _tc_layer_call(False, False, (
        bg_c, cnts[0][0], cnts[0][1], Wl_1_cause, biases1[0],
        bg_r, cnts[1][0], cnts[1][1], Wl_1_relate, biases1[1],
        bg_ch, cnts[2][0], cnts[2][1], Wl_1_child, biases1[2],
        Wr_1_cause, Wr_1_relate, Wr_1_child, hd2,
        bg_v, cnts[3][0], cnts[3][1], Wl_1_rev, biases1[3], Wr_1_rev, hc2,
    ))
    return out_c, out_d


# speedup vs baseline: 1.3176x; 1.0027x over previous
import functools

import jax
import jax.numpy as jnp
from jax import lax
from jax.experimental import pallas as pl
from jax.experimental.pallas import tpu as pltpu
from jax.experimental.pallas import tpu_sc as plsc

N = 10000
E = 40000
D = 256
H = 128
NS = 16
NC = 2
EPT = 2560
EPAD = NS * EPT
CH = 128
NCHUNK = EPT // CH
ACC_ROWS = 10112
STRIPE = ACC_ROWS // NS
TRASH = 10100
EPT_CNT = EPAD // (NC * NS)
CNT_CH = 128

_MESH = plsc.VectorSubcoreMesh(core_axis_name="c", subcore_axis_name="s")


@functools.partial(
    pl.kernel,
    out_type=tuple([jax.ShapeDtypeStruct((NC, ACC_ROWS, H), jnp.float32)] * 4),
    mesh=_MESH,
    scratch_types=[
        pltpu.VMEM_SHARED((ACC_ROWS, H), jnp.float32),
        pltpu.VMEM((NCHUNK, CH), jnp.int32),
        pltpu.VMEM((NCHUNK, CH), jnp.int32),
        pltpu.VMEM((CH, H), jnp.float32),
        pltpu.SemaphoreType.DMA,
    ])
def _sc_agg(tab_c, tab_d,
            src_cause, dst_cause, src_relate, dst_relate,
            src_child, dst_child, src_rev, dst_rev,
            zrow_hbm,
            out_cause, out_relate, out_child, out_rev,
            acc_sh, sidx_st, didx_st, rows_v, sem):
    tile = lax.axis_index("s")
    cid = lax.axis_index("c")

    rels = (
        (src_cause, dst_cause, tab_c, out_cause),
        (src_relate, dst_relate, tab_c, out_relate),
        (src_child, dst_child, tab_d, out_child),
        (src_rev, dst_rev, tab_d, out_rev),
    )
    for (src3, dst3, tab, agg_out) in rels:
        d1 = pltpu.async_copy(
            zrow_hbm, acc_sh.at[pl.ds(tile * STRIPE, STRIPE)], sem)
        d2 = pltpu.async_copy(src3.at[cid, tile], sidx_st, sem)
        d3 = pltpu.async_copy(dst3.at[tile], didx_st, sem)
        d1.wait()
        d2.wait()
        d3.wait()
        plsc.subcore_barrier()

        def chunk(j, carry):
            pltpu.async_copy(tab.at[sidx_st.at[j]], rows_v, sem).wait()
            pltpu.sync_copy(rows_v, acc_sh.at[didx_st.at[j]], add=True)
            return carry

        lax.fori_loop(0, NCHUNK, chunk, 0)
        plsc.subcore_barrier()
        pltpu.sync_copy(acc_sh.at[pl.ds(tile * STRIPE, STRIPE)],
                        agg_out.at[cid, pl.ds(tile * STRIPE, STRIPE)])


@functools.partial(
    pl.kernel,
    out_type=tuple([jax.ShapeDtypeStruct((NC, ACC_ROWS, H), jnp.float32)] * 4),
    mesh=_MESH,
    scratch_types=[
        pltpu.VMEM_SHARED((ACC_ROWS, H), jnp.float32),
        pltpu.VMEM((EPT_CNT // CNT_CH, CNT_CH), jnp.int32),
        pltpu.VMEM((CNT_CH, H), jnp.float32),
    ])
def _sc_counts(dst_cause, dst_relate, dst_child, dst_rev,
               zrow_hbm, ones_hbm,
               out_cause, out_relate, out_child, out_rev,
               cnt_sh, didx_st, ones_v):
    tile = lax.axis_index("s")
    cid = lax.axis_index("c")

    pltpu.sync_copy(ones_hbm, ones_v)
    for (dst4, cnt_out) in ((dst_cause, out_cause), (dst_relate, out_relate),
                            (dst_child, out_child), (dst_rev, out_rev)):
        pltpu.sync_copy(zrow_hbm, cnt_sh.at[pl.ds(tile * STRIPE, STRIPE)])
        pltpu.sync_copy(dst4.at[cid, tile], didx_st)
        plsc.subcore_barrier()

        def chunk(j, carry):
            pltpu.sync_copy(ones_v, cnt_sh.at[didx_st.at[j]], add=True)
            return carry

        lax.fori_loop(0, EPT_CNT // CNT_CH, chunk, 0)
        plsc.subcore_barrier()
        pltpu.sync_copy(cnt_sh.at[pl.ds(tile * STRIPE, STRIPE)],
                        cnt_out.at[cid, pl.ds(tile * STRIPE, STRIPE)])


def _dot_t(x, w):
    return lax.dot_general(x, w, (((1,), (1,)), ((), ())),
                           preferred_element_type=jnp.float32)


def _tc_layer_body(relu, halves,
                   aggc, cntc_a, cntc_b, wlc, blc,
                   aggr, cntr_a, cntr_b, wlr, blr,
                   aggch, cntch_a, cntch_b, wlch, blch,
                   wrc, wrr, wrch, xd2,
                   aggv, cntv_a, cntv_b, wlv, blv, wrv, xc2,
                   outc_ref, outd_ref):
    def sage(agg_ref, cnt_a, cnt_b, wl_ref):
        cnt = cnt_a[:, 0:1] + cnt_b[:, 0:1]
        inv = 1.0 / jnp.maximum(cnt, 1.0)
        lo = agg_ref[0] * inv
        hi = agg_ref[1] * inv
        return _dot_t(lo, wl_ref[:, 0:H]) + _dot_t(hi, wl_ref[:, H:D])

    def dense(x_ref, w):
        return _dot_t(x_ref[0], w[:, 0:H]) + _dot_t(x_ref[1], w[:, H:D])

    outd = (sage(aggc, cntc_a, cntc_b, wlc)
            + sage(aggr, cntr_a, cntr_b, wlr)
            + sage(aggch, cntch_a, cntch_b, wlch)
            + dense(xd2, wrc[...] + wrr[...] + wrch[...])
            + (blc[...] + blr[...] + blch[...]))
    outc = sage(aggv, cntv_a, cntv_b, wlv) + dense(xc2, wrv[...]) + blv[...]
    if relu:
        outd = jnp.maximum(outd, 0.0)
        outc = jnp.maximum(outc, 0.0)
    if halves:
        outc_ref[0] = outc[:, 0:H]
        outc_ref[1] = outc[:, H:D]
        outd_ref[0] = outd[:, 0:H]
        outd_ref[1] = outd[:, H:D]
    else:
        outc_ref[...] = outc
        outd_ref[...] = outd


def _tc_layer_call(relu, halves, args):
    MB = 400
    grid = (N // MB,)
    agg_spec = pl.BlockSpec((NC, MB, H), lambda i: (0, i, 0))
    cnt_spec = pl.BlockSpec((MB, 16), lambda i: (i, 0))
    w_spec = pl.BlockSpec((D, D), lambda i: (0, 0))
    b_spec = pl.BlockSpec((1, D), lambda i: (0, 0))
    x_spec = pl.BlockSpec((NC, MB, H), lambda i: (0, i, 0))
    rel_spec = [agg_spec, cnt_spec, cnt_spec, w_spec, b_spec]
    in_specs = (rel_spec * 3
                + [w_spec, w_spec, w_spec, x_spec]
                + rel_spec + [w_spec, x_spec])
    if halves:
        out_shape = [jax.ShapeDtypeStruct((NC, N, H), jnp.float32)] * 2
        out_specs = [pl.BlockSpec((NC, MB, H), lambda i: (0, i, 0))] * 2
    else:
        out_shape = [jax.ShapeDtypeStruct((N, D), jnp.float32)] * 2
        out_specs = [pl.BlockSpec((MB, D), lambda i: (i, 0))] * 2
    fn = pl.pallas_call(
        functools.partial(_tc_layer_body, relu, halves),
        grid=grid, in_specs=in_specs, out_specs=out_specs,
        out_shape=out_shape)
    return fn(*args)


def kernel(x_chemical, x_disease,
           edge_index_cause, edge_index_relate, edge_index_rev,
           edge_index_child,
           Wl_0_cause, bl_0_cause, Wr_0_cause,
           Wl_0_relate, bl_0_relate, Wr_0_relate,
           Wl_0_rev, bl_0_rev, Wr_0_rev,
           Wl_0_child, bl_0_child, Wr_0_child,
           Wl_1_cause, bl_1_cause, Wr_1_cause,
           Wl_1_relate, bl_1_relate, Wr_1_relate,
           Wl_1_rev, bl_1_rev, Wr_1_rev,
           Wl_1_child, bl_1_child, Wr_1_child):
    def stack_halves(x):
        return x.reshape(N, NC, H).transpose(1, 0, 2)

    xc2 = stack_halves(x_chemical)
    xd2 = stack_halves(x_disease)

    def prep_edges(ei):
        src = ei[0].astype(jnp.int32)
        dst = ei[1].astype(jnp.int32)
        pad = EPAD - E
        src = jnp.concatenate([src, jnp.zeros((pad,), jnp.int32)])
        dst = jnp.concatenate([dst, jnp.full((pad,), TRASH, jnp.int32)])
        src3 = src.reshape(NS, NCHUNK, CH)
        src3 = jnp.stack([src3, src3 + N])
        dst3 = dst.reshape(NS, NCHUNK, CH)
        dst4 = dst.reshape(NC, NS, EPT_CNT // CNT_CH, CNT_CH)
        return src3, dst3, dst4

    sc_c, dc3, dc = prep_edges(edge_index_cause)
    sc_r, dr3, dr = prep_edges(edge_index_relate)
    sc_ch, dch3, dch = prep_edges(edge_index_child)
    sc_v, dv3, dv = prep_edges(edge_index_rev)

    zrow = jnp.zeros((STRIPE, H), jnp.float32)
    ones = jnp.ones((CNT_CH, H), jnp.float32)

    biases0 = [b.reshape(1, D) for b in
               (bl_0_cause, bl_0_relate, bl_0_child, bl_0_rev)]
    biases1 = [b.reshape(1, D) for b in
               (bl_1_cause, bl_1_relate, bl_1_child, bl_1_rev)]

    cnt2 = _sc_counts(dc, dr, dch, dv, zrow, ones)
    cnts = [(c2[0, :, 0:16], c2[1, :, 0:16]) for c2 in cnt2]

    (ag_c, ag_r, ag_ch, ag_v) = _sc_agg(
        xc2.reshape(NC * N, H), xd2.reshape(NC * N, H),
        sc_c, dc3, sc_r, dr3, sc_ch, dch3, sc_v, dv3, zrow)

    hc2, hd2 = _tc_layer_call(True, True, (
        ag_c, cnts[0][0], cnts[0][1], Wl_0_cause, biases0[0],
        ag_r, cnts[1][0], cnts[1][1], Wl_0_relate, biases0[1],
        ag_ch, cnts[2][0], cnts[2][1], Wl_0_child, biases0[2],
        Wr_0_cause, Wr_0_relate, Wr_0_child, xd2,
        ag_v, cnts[3][0], cnts[3][1], Wl_0_rev, biases0[3], Wr_0_rev, xc2,
    ))

    (bg_c, bg_r, bg_ch, bg_v) = _sc_agg(
        hc2.reshape(NC * N, H), hd2.reshape(NC * N, H),
        sc_c, dc3, sc_r, dr3, sc_ch, dch3, sc_v, dv3, zrow)

    out_c, out_d = _tc_layer_call(False, False, (
        bg_c, cnts[0][0], cnts[0][1], Wl_1_cause, biases1[0],
        bg_r, cnts[1][0], cnts[1][1], Wl_1_relate, biases1[1],
        bg_ch, cnts[2][0], cnts[2][1], Wl_1_child, biases1[2],
        Wr_1_cause, Wr_1_relate, Wr_1_child, hd2,
        bg_v, cnts[3][0], cnts[3][1], Wl_1_rev, biases1[3], Wr_1_rev, hc2,
    ))
    return out_c, out_d


# speedup vs baseline: 1.3390x; 1.0162x over previous
import functools

import jax
import jax.numpy as jnp
from jax import lax
from jax.experimental import pallas as pl
from jax.experimental.pallas import tpu as pltpu
from jax.experimental.pallas import tpu_sc as plsc

N = 10000
E = 40000
D = 256
H = 128
NS = 16
NC = 2
EPT = 2560
EPAD = NS * EPT
CH = 64
NCHUNK = EPT // CH
ACC_ROWS = 10112
STRIPE = ACC_ROWS // NS
TRASH = 10100
EPT_CNT = EPAD // (NC * NS)
CNT_CH = 128

_MESH = plsc.VectorSubcoreMesh(core_axis_name="c", subcore_axis_name="s")


@functools.partial(
    pl.kernel,
    out_type=tuple([jax.ShapeDtypeStruct((NC, ACC_ROWS, H), jnp.float32)] * 4),
    mesh=_MESH,
    scratch_types=[
        pltpu.VMEM_SHARED((ACC_ROWS, H), jnp.float32),
        pltpu.VMEM((NCHUNK, CH), jnp.int32),
        pltpu.VMEM((NCHUNK, CH), jnp.int32),
        pltpu.VMEM((CH, H), jnp.float32),
        pltpu.VMEM((CH, H), jnp.float32),
        pltpu.SemaphoreType.DMA,
        pltpu.SemaphoreType.DMA,
        pltpu.SemaphoreType.DMA,
    ])
def _sc_agg(tab_c, tab_d,
            src_cause, dst_cause, src_relate, dst_relate,
            src_child, dst_child, src_rev, dst_rev,
            zrow_hbm,
            out_cause, out_relate, out_child, out_rev,
            acc_sh, sidx_st, didx_st, rows_a, rows_b, sem, sem_a, sem_b):
    tile = lax.axis_index("s")
    cid = lax.axis_index("c")

    rels = (
        (src_cause, dst_cause, tab_c, out_cause),
        (src_relate, dst_relate, tab_c, out_relate),
        (src_child, dst_child, tab_d, out_child),
        (src_rev, dst_rev, tab_d, out_rev),
    )
    for (src3, dst3, tab, agg_out) in rels:
        d1 = pltpu.async_copy(
            zrow_hbm, acc_sh.at[pl.ds(tile * STRIPE, STRIPE)], sem)
        d2 = pltpu.async_copy(src3.at[cid, tile], sidx_st, sem)
        d3 = pltpu.async_copy(dst3.at[tile], didx_st, sem)
        d1.wait()
        d2.wait()
        d3.wait()
        plsc.subcore_barrier()

        pltpu.async_copy(tab.at[sidx_st.at[0]], rows_a, sem_a)

        def pair(t, carry):
            j0 = 2 * t
            pltpu.make_async_copy(tab.at[sidx_st.at[j0]], rows_a,
                                  sem_a).wait()
            pltpu.async_copy(tab.at[sidx_st.at[j0 + 1]], rows_b, sem_b)
            pltpu.sync_copy(rows_a, acc_sh.at[didx_st.at[j0]], add=True)
            pltpu.make_async_copy(tab.at[sidx_st.at[j0 + 1]], rows_b,
                                  sem_b).wait()
            pltpu.async_copy(tab.at[sidx_st.at[j0 + 2]], rows_a, sem_a)
            pltpu.sync_copy(rows_b, acc_sh.at[didx_st.at[j0 + 1]], add=True)
            return carry

        lax.fori_loop(0, NCHUNK // 2 - 1, pair, 0)
        pltpu.make_async_copy(tab.at[sidx_st.at[NCHUNK - 2]], rows_a,
                              sem_a).wait()
        pltpu.async_copy(tab.at[sidx_st.at[NCHUNK - 1]], rows_b, sem_b)
        pltpu.sync_copy(rows_a, acc_sh.at[didx_st.at[NCHUNK - 2]], add=True)
        pltpu.make_async_copy(tab.at[sidx_st.at[NCHUNK - 1]], rows_b,
                              sem_b).wait()
        pltpu.sync_copy(rows_b, acc_sh.at[didx_st.at[NCHUNK - 1]], add=True)
        plsc.subcore_barrier()
        pltpu.sync_copy(acc_sh.at[pl.ds(tile * STRIPE, STRIPE)],
                        agg_out.at[cid, pl.ds(tile * STRIPE, STRIPE)])


@functools.partial(
    pl.kernel,
    out_type=tuple([jax.ShapeDtypeStruct((NC, ACC_ROWS, H), jnp.float32)] * 4),
    mesh=_MESH,
    scratch_types=[
        pltpu.VMEM_SHARED((ACC_ROWS, H), jnp.float32),
        pltpu.VMEM((EPT_CNT // CNT_CH, CNT_CH), jnp.int32),
        pltpu.VMEM((CNT_CH, H), jnp.float32),
    ])
def _sc_counts(dst_cause, dst_relate, dst_child, dst_rev,
               zrow_hbm, ones_hbm,
               out_cause, out_relate, out_child, out_rev,
               cnt_sh, didx_st, ones_v):
    tile = lax.axis_index("s")
    cid = lax.axis_index("c")

    pltpu.sync_copy(ones_hbm, ones_v)
    for (dst4, cnt_out) in ((dst_cause, out_cause), (dst_relate, out_relate),
                            (dst_child, out_child), (dst_rev, out_rev)):
        pltpu.sync_copy(zrow_hbm, cnt_sh.at[pl.ds(tile * STRIPE, STRIPE)])
        pltpu.sync_copy(dst4.at[cid, tile], didx_st)
        plsc.subcore_barrier()

        def chunk(j, carry):
            pltpu.sync_copy(ones_v, cnt_sh.at[didx_st.at[j]], add=True)
            return carry

        lax.fori_loop(0, EPT_CNT // CNT_CH, chunk, 0)
        plsc.subcore_barrier()
        pltpu.sync_copy(cnt_sh.at[pl.ds(tile * STRIPE, STRIPE)],
                        cnt_out.at[cid, pl.ds(tile * STRIPE, STRIPE)])


def _dot_t(x, w):
    return lax.dot_general(x, w, (((1,), (1,)), ((), ())),
                           preferred_element_type=jnp.float32)


def _tc_layer_body(relu, halves,
                   aggc, cntc_a, cntc_b, wlc, blc,
                   aggr, cntr_a, cntr_b, wlr, blr,
                   aggch, cntch_a, cntch_b, wlch, blch,
                   wrc, wrr, wrch, xd2,
                   aggv, cntv_a, cntv_b, wlv, blv, wrv, xc2,
                   outc_ref, outd_ref):
    def sage(agg_ref, cnt_a, cnt_b, wl_ref):
        cnt = cnt_a[:, 0:1] + cnt_b[:, 0:1]
        inv = 1.0 / jnp.maximum(cnt, 1.0)
        lo = agg_ref[0] * inv
        hi = agg_ref[1] * inv
        return _dot_t(lo, wl_ref[:, 0:H]) + _dot_t(hi, wl_ref[:, H:D])

    def dense(x_ref, w):
        return _dot_t(x_ref[0], w[:, 0:H]) + _dot_t(x_ref[1], w[:, H:D])

    outd = (sage(aggc, cntc_a, cntc_b, wlc)
            + sage(aggr, cntr_a, cntr_b, wlr)
            + sage(aggch, cntch_a, cntch_b, wlch)
            + dense(xd2, wrc[...] + wrr[...] + wrch[...])
            + (blc[...] + blr[...] + blch[...]))
    outc = sage(aggv, cntv_a, cntv_b, wlv) + dense(xc2, wrv[...]) + blv[...]
    if relu:
        outd = jnp.maximum(outd, 0.0)
        outc = jnp.maximum(outc, 0.0)
    if halves:
        outc_ref[0] = outc[:, 0:H]
        outc_ref[1] = outc[:, H:D]
        outd_ref[0] = outd[:, 0:H]
        outd_ref[1] = outd[:, H:D]
    else:
        outc_ref[...] = outc
        outd_ref[...] = outd


def _tc_layer_call(relu, halves, args):
    MB = 400
    grid = (N // MB,)
    agg_spec = pl.BlockSpec((NC, MB, H), lambda i: (0, i, 0))
    cnt_spec = pl.BlockSpec((MB, 16), lambda i: (i, 0))
    w_spec = pl.BlockSpec((D, D), lambda i: (0, 0))
    b_spec = pl.BlockSpec((1, D), lambda i: (0, 0))
    x_spec = pl.BlockSpec((NC, MB, H), lambda i: (0, i, 0))
    rel_spec = [agg_spec, cnt_spec, cnt_spec, w_spec, b_spec]
    in_specs = (rel_spec * 3
                + [w_spec, w_spec, w_spec, x_spec]
                + rel_spec + [w_spec, x_spec])
    if halves:
        out_shape = [jax.ShapeDtypeStruct((NC, N, H), jnp.float32)] * 2
        out_specs = [pl.BlockSpec((NC, MB, H), lambda i: (0, i, 0))] * 2
    else:
        out_shape = [jax.ShapeDtypeStruct((N, D), jnp.float32)] * 2
        out_specs = [pl.BlockSpec((MB, D), lambda i: (i, 0))] * 2
    fn = pl.pallas_call(
        functools.partial(_tc_layer_body, relu, halves),
        grid=grid, in_specs=in_specs, out_specs=out_specs,
        out_shape=out_shape)
    return fn(*args)


def kernel(x_chemical, x_disease,
           edge_index_cause, edge_index_relate, edge_index_rev,
           edge_index_child,
           Wl_0_cause, bl_0_cause, Wr_0_cause,
           Wl_0_relate, bl_0_relate, Wr_0_relate,
           Wl_0_rev, bl_0_rev, Wr_0_rev,
           Wl_0_child, bl_0_child, Wr_0_child,
           Wl_1_cause, bl_1_cause, Wr_1_cause,
           Wl_1_relate, bl_1_relate, Wr_1_relate,
           Wl_1_rev, bl_1_rev, Wr_1_rev,
           Wl_1_child, bl_1_child, Wr_1_child):
    def stack_halves(x):
        return x.reshape(N, NC, H).transpose(1, 0, 2)

    xc2 = stack_halves(x_chemical)
    xd2 = stack_halves(x_disease)

    def prep_edges(ei):
        src = ei[0].astype(jnp.int32)
        dst = ei[1].astype(jnp.int32)
        pad = EPAD - E
        src = jnp.concatenate([src, jnp.zeros((pad,), jnp.int32)])
        dst = jnp.concatenate([dst, jnp.full((pad,), TRASH, jnp.int32)])
        src3 = src.reshape(NS, NCHUNK, CH)
        src3 = jnp.stack([src3, src3 + N])
        dst3 = dst.reshape(NS, NCHUNK, CH)
        dst4 = dst.reshape(NC, NS, EPT_CNT // CNT_CH, CNT_CH)
        return src3, dst3, dst4

    sc_c, dc3, dc = prep_edges(edge_index_cause)
    sc_r, dr3, dr = prep_edges(edge_index_relate)
    sc_ch, dch3, dch = prep_edges(edge_index_child)
    sc_v, dv3, dv = prep_edges(edge_index_rev)

    zrow = jnp.zeros((STRIPE, H), jnp.float32)
    ones = jnp.ones((CNT_CH, H), jnp.float32)

    biases0 = [b.reshape(1, D) for b in
               (bl_0_cause, bl_0_relate, bl_0_child, bl_0_rev)]
    biases1 = [b.reshape(1, D) for b in
               (bl_1_cause, bl_1_relate, bl_1_child, bl_1_rev)]

    cnt2 = _sc_counts(dc, dr, dch, dv, zrow, ones)
    cnts = [(c2[0, :, 0:16], c2[1, :, 0:16]) for c2 in cnt2]

    (ag_c, ag_r, ag_ch, ag_v) = _sc_agg(
        xc2.reshape(NC * N, H), xd2.reshape(NC * N, H),
        sc_c, dc3, sc_r, dr3, sc_ch, dch3, sc_v, dv3, zrow)

    hc2, hd2 = _tc_layer_call(True, True, (
        ag_c, cnts[0][0], cnts[0][1], Wl_0_cause, biases0[0],
        ag_r, cnts[1][0], cnts[1][1], Wl_0_relate, biases0[1],
        ag_ch, cnts[2][0], cnts[2][1], Wl_0_child, biases0[2],
        Wr_0_cause, Wr_0_relate, Wr_0_child, xd2,
        ag_v, cnts[3][0], cnts[3][1], Wl_0_rev, biases0[3], Wr_0_rev, xc2,
    ))

    (bg_c, bg_r, bg_ch, bg_v) = _sc_agg(
        hc2.reshape(NC * N, H), hd2.reshape(NC * N, H),
        sc_c, dc3, sc_r, dr3, sc_ch, dch3, sc_v, dv3, zrow)

    out_c, out_d = _tc_layer_call(False, False, (
        bg_c, cnts[0][0], cnts[0][1], Wl_1_cause, biases1[0],
        bg_r, cnts[1][0], cnts[1][1], Wl_1_relate, biases1[1],
        bg_ch, cnts[2][0], cnts[2][1], Wl_1_child, biases1[2],
        Wr_1_cause, Wr_1_relate, Wr_1_child, hd2,
        bg_v, cnts[3][0], cnts[3][1], Wl_1_rev, biases1[3], Wr_1_rev, hc2,
    ))
    return out_c, out_d
